# baseline (speedup 1.0000x reference)
import functools

import numpy as np

import jax
import jax.numpy as jnp
from jax import lax
from jax.experimental import pallas as pl
from jax.experimental.pallas import tpu as pltpu
from jax.experimental.pallas import tpu_sc as plsc

_NC = 2
_NS = 16
_NW = _NC * _NS
_LANES = 16
_NIN = 8
_NMF = 3
_HTBL = 96
_TILE = 1664
_LANE_TILE = 128


def _pack4(i0, i1, i2, i3):
    return ((i0 * _NMF + i1) * _NMF + i2) * _NMF + i3


def _sc_body(n_aligned, rows, xf_hbm, idx_hbm, addr_hbm, out_hbm,
             xrows, atbl, btbl, addrbuf, idxtile, hitile, lotile, bufs,
             outsems, idxsem):
    cid = lax.axis_index("c")
    sid = lax.axis_index("s")
    wid = sid * _NC + cid
    base = wid * rows

    tiles = []
    r0 = 0
    while r0 < n_aligned:
        w = min(_TILE, n_aligned - r0)
        tiles.append((r0, w, w))
        r0 += w

    pltpu.sync_copy(xf_hbm.at[pl.ds(base * (_NIN * _NMF), rows * _NIN * _NMF)],
                    xrows)

    pltpu.sync_copy(addr_hbm, addrbuf)

    def build_tables(bl, carry):
        xrow = xrows.at[pl.ds(bl * (_NIN * _NMF), _NIN * _NMF)]
        for c in range(_HTBL // _LANES):
            for t, tbl in enumerate((atbl, btbl)):
                vs = []
                for i in range(4):
                    av = addrbuf[pl.ds(((c * 2 + t) * 4 + i) * _LANES,
                                       _LANES)]
                    vs.append(plsc.load_gather(xrow, [av]))
                tbl[pl.ds(bl * _HTBL + c * _LANES, _LANES)] = \
                    (vs[0] * vs[1]) * (vs[2] * vs[3])
        return carry

    lax.fori_loop(0, rows, build_tables, None)

    pending = [None, None]
    idx_pending = pltpu.async_copy(idx_hbm.at[:, pl.ds(0, tiles[0][1])],
                                   idxtile.at[:, pl.ds(0, tiles[0][1])],
                                   idxsem)
    for t, (r0, w, wc) in enumerate(tiles):
        buf = bufs[t % 2]
        sem = outsems[t % 2]
        if pending[t % 2] is not None:
            pending[t % 2].wait()
            pending[t % 2] = None
        idx_pending.wait()

        @plsc.parallel_loop(0, wc, step=_LANES, unroll=2)
        def pack_codes(off):
            iv = [idxtile[i, pl.ds(off, _LANES)] for i in range(_NIN)]
            hitile[pl.ds(off, _LANES)] = _pack4(iv[0], iv[1], iv[2], iv[3])
            lotile[pl.ds(off, _LANES)] = _pack4(iv[4], iv[5], iv[6], iv[7])

        if t + 1 < len(tiles):
            nw = tiles[t + 1][1]
            idx_pending = pltpu.async_copy(
                idx_hbm.at[:, pl.ds(r0 + w, nw)],
                idxtile.at[:, pl.ds(0, nw)], idxsem)

        @plsc.parallel_loop(0, wc, step=_LANES, unroll=2)
        def gather_mul(off):
            hi = hitile[pl.ds(off, _LANES)]
            lo = lotile[pl.ds(off, _LANES)]
            for bl in range(rows):
                av = plsc.load_gather(atbl.at[pl.ds(bl * _HTBL, _HTBL)], [hi])
                bv = plsc.load_gather(btbl.at[pl.ds(bl * _HTBL, _HTBL)], [lo])
                buf[bl, pl.ds(off, _LANES)] = av * bv

        pending[t % 2] = pltpu.async_copy(
            buf.at[:, pl.ds(0, w)],
            out_hbm.at[pl.ds(base, rows), pl.ds(r0, w)], sem)

    for p in pending:
        if p is not None:
            p.wait()


def _tc_tail_block(x_ref, idx_ref, aliased_ref, o_ref):
    acc = None
    for i in range(_NIN):
        idx = idx_ref[i : i + 1, :]
        x0 = x_ref[:, 3 * i : 3 * i + 1]
        x1 = x_ref[:, 3 * i + 1 : 3 * i + 2]
        x2 = x_ref[:, 3 * i + 2 : 3 * i + 3]
        v = jnp.where(idx == 0, x0, jnp.where(idx == 1, x1, x2))
        acc = v if acc is None else acc * v
    o_ref[:, :] = acc


def kernel(x, mf_indices):
    B, n_in, n_mfs = x.shape
    n_rules = mf_indices.shape[0]
    rows = B // _NW
    n_aligned = (n_rules // _LANE_TILE) * _LANE_TILE
    tail_blk = n_aligned // _LANE_TILE
    xf2d = x.reshape(B, n_in * n_mfs)
    xf = xf2d.reshape(B * n_in * n_mfs)
    idxT = mf_indices.astype(jnp.int32).T

    a = np.arange(_HTBL)
    digs = [(a // 27) % _NMF, (a // 9) % _NMF, (a // _NMF) % _NMF, a % _NMF]
    addr_np = np.empty((_HTBL // _LANES, 2, 4, _LANES), dtype=np.int32)
    for c in range(_HTBL // _LANES):
        sl = slice(c * _LANES, (c + 1) * _LANES)
        for t in range(2):
            for i in range(4):
                addr_np[c, t, i] = t * 12 + 3 * i + digs[i][sl]
    addrs = jnp.asarray(addr_np.reshape(-1))

    mesh = plsc.VectorSubcoreMesh(core_axis_name="c", subcore_axis_name="s")
    body = functools.partial(_sc_body, n_aligned, rows)
    sc_fill = pl.kernel(
        body,
        out_type=jax.ShapeDtypeStruct((B, n_rules), jnp.float32),
        mesh=mesh,
        scratch_types=dict(
            xrows=pltpu.VMEM((rows * n_in * n_mfs,), jnp.float32),
            atbl=pltpu.VMEM((rows * _HTBL,), jnp.float32),
            btbl=pltpu.VMEM((rows * _HTBL,), jnp.float32),
            addrbuf=pltpu.VMEM((2 * 4 * _HTBL,), jnp.int32),
            idxtile=pltpu.VMEM((n_in, _TILE), jnp.int32),
            hitile=pltpu.VMEM((_TILE,), jnp.int32),
            lotile=pltpu.VMEM((_TILE,), jnp.int32),
            bufs=[pltpu.VMEM((rows, _TILE), jnp.float32) for _ in range(2)],
            outsems=[pltpu.SemaphoreType.DMA for _ in range(2)],
            idxsem=pltpu.SemaphoreType.DMA,
        ),
        compiler_params=pltpu.CompilerParams(
            needs_layout_passes=False, disable_bounds_checks=True),
    )
    out_main = sc_fill(xf, idxT, addrs)

    return pl.pallas_call(
        _tc_tail_block,
        grid=(1,),
        in_specs=[
            pl.BlockSpec((B, n_in * n_mfs), lambda j: (0, 0)),
            pl.BlockSpec((n_in, _LANE_TILE), lambda j: (0, tail_blk)),
            pl.BlockSpec(memory_space=pl.ANY),
        ],
        out_specs=pl.BlockSpec((B, _LANE_TILE), lambda j: (0, tail_blk)),
        out_shape=jax.ShapeDtypeStruct((B, n_rules), jnp.float32),
        input_output_aliases={2: 0},
    )(xf2d, idxT, out_main)


# speedup vs baseline: 1.0491x; 1.0491x over previous
import functools

import numpy as np

import jax
import jax.numpy as jnp
from jax import lax
from jax.experimental import pallas as pl
from jax.experimental.pallas import tpu as pltpu
from jax.experimental.pallas import tpu_sc as plsc

_NC = 2
_NS = 16
_NW = _NC * _NS
_LANES = 16
_NIN = 8
_NMF = 3
_HTBL = 96
_TILE = 1664
_LANE_TILE = 128


def _pack4(i0, i1, i2, i3):
    return ((i0 * _NMF + i1) * _NMF + i2) * _NMF + i3


def _sc_body(n_aligned, rows, xf_hbm, idx_hbm, addr_hbm, out_hbm,
             xrows, atbl, btbl, addrbuf, idxtile, hitile, lotile, bufs,
             outsems, idxsem):
    cid = lax.axis_index("c")
    sid = lax.axis_index("s")
    wid = sid * _NC + cid
    base = wid * rows

    tiles = []
    r0 = 0
    while r0 < n_aligned:
        w = min(_TILE, n_aligned - r0)
        tiles.append((r0, w, w))
        r0 += w

    pltpu.sync_copy(xf_hbm.at[pl.ds(base * (_NIN * _NMF), rows * _NIN * _NMF)],
                    xrows)

    pltpu.sync_copy(addr_hbm, addrbuf)

    def build_tables(bl, carry):
        xrow = xrows.at[pl.ds(bl * (_NIN * _NMF), _NIN * _NMF)]
        for c in range(_HTBL // _LANES):
            for t, tbl in enumerate((atbl, btbl)):
                vs = []
                for i in range(4):
                    av = addrbuf[pl.ds(((c * 2 + t) * 4 + i) * _LANES,
                                       _LANES)]
                    vs.append(plsc.load_gather(xrow, [av]))
                tbl[pl.ds(bl * _HTBL + c * _LANES, _LANES)] = \
                    (vs[0] * vs[1]) * (vs[2] * vs[3])
        return carry

    lax.fori_loop(0, rows, build_tables, None)

    pending = [None, None]
    idx_pending = pltpu.async_copy(idx_hbm.at[:, pl.ds(0, tiles[0][1])],
                                   idxtile.at[:, pl.ds(0, tiles[0][1])],
                                   idxsem)
    for t, (r0, w, wc) in enumerate(tiles):
        buf = bufs[t % 2]
        sem = outsems[t % 2]
        if pending[t % 2] is not None:
            pending[t % 2].wait()
            pending[t % 2] = None
        idx_pending.wait()

        @plsc.parallel_loop(0, wc, step=_LANES, unroll=2)
        def pack_codes(off):
            iv = [idxtile[i, pl.ds(off, _LANES)] for i in range(_NIN)]
            hitile[pl.ds(off, _LANES)] = _pack4(iv[0], iv[1], iv[2], iv[3])
            lotile[pl.ds(off, _LANES)] = _pack4(iv[4], iv[5], iv[6], iv[7])

        if t + 1 < len(tiles):
            nw = tiles[t + 1][1]
            idx_pending = pltpu.async_copy(
                idx_hbm.at[:, pl.ds(r0 + w, nw)],
                idxtile.at[:, pl.ds(0, nw)], idxsem)

        @plsc.parallel_loop(0, wc, step=_LANES, unroll=2)
        def gather_mul(off):
            hi = hitile[pl.ds(off, _LANES)]
            lo = lotile[pl.ds(off, _LANES)]
            for bl in range(rows):
                av = plsc.load_gather(atbl.at[pl.ds(bl * _HTBL, _HTBL)], [hi])
                bv = plsc.load_gather(btbl.at[pl.ds(bl * _HTBL, _HTBL)], [lo])
                buf[bl, pl.ds(off, _LANES)] = av * bv

        pending[t % 2] = pltpu.async_copy(
            buf.at[:, pl.ds(0, w)],
            out_hbm.at[pl.ds(base, rows), pl.ds(r0, w)], sem)

    for p in pending:
        if p is not None:
            p.wait()


def _tc_tail_block(x_ref, idx_ref, aliased_ref, o_ref):
    acc = None
    for i in range(_NIN):
        idx = idx_ref[i : i + 1, :]
        x0 = x_ref[:, 3 * i : 3 * i + 1]
        x1 = x_ref[:, 3 * i + 1 : 3 * i + 2]
        x2 = x_ref[:, 3 * i + 2 : 3 * i + 3]
        v = jnp.where(idx == 0, x0, jnp.where(idx == 1, x1, x2))
        acc = v if acc is None else acc * v
    o_ref[:, :] = acc


def kernel(x, mf_indices):
    B, n_in, n_mfs = x.shape
    n_rules = mf_indices.shape[0]
    rows = B // _NW
    n_aligned = pl.cdiv(n_rules, _TILE) * _TILE
    xf2d = x.reshape(B, n_in * n_mfs)
    xf = xf2d.reshape(B * n_in * n_mfs)
    idxT = mf_indices.astype(jnp.int32).T
    idxT = jnp.pad(idxT, ((0, 0), (0, n_aligned - n_rules)))

    a = np.arange(_HTBL)
    digs = [(a // 27) % _NMF, (a // 9) % _NMF, (a // _NMF) % _NMF, a % _NMF]
    addr_np = np.empty((_HTBL // _LANES, 2, 4, _LANES), dtype=np.int32)
    for c in range(_HTBL // _LANES):
        sl = slice(c * _LANES, (c + 1) * _LANES)
        for t in range(2):
            for i in range(4):
                addr_np[c, t, i] = t * 12 + 3 * i + digs[i][sl]
    addrs = jnp.asarray(addr_np.reshape(-1))

    mesh = plsc.VectorSubcoreMesh(core_axis_name="c", subcore_axis_name="s")
    body = functools.partial(_sc_body, n_aligned, rows)
    sc_fill = pl.kernel(
        body,
        out_type=jax.ShapeDtypeStruct((B, n_aligned), jnp.float32),
        mesh=mesh,
        scratch_types=dict(
            xrows=pltpu.VMEM((rows * n_in * n_mfs,), jnp.float32),
            atbl=pltpu.VMEM((rows * _HTBL,), jnp.float32),
            btbl=pltpu.VMEM((rows * _HTBL,), jnp.float32),
            addrbuf=pltpu.VMEM((2 * 4 * _HTBL,), jnp.int32),
            idxtile=pltpu.VMEM((n_in, _TILE), jnp.int32),
            hitile=pltpu.VMEM((_TILE,), jnp.int32),
            lotile=pltpu.VMEM((_TILE,), jnp.int32),
            bufs=[pltpu.VMEM((rows, _TILE), jnp.float32) for _ in range(2)],
            outsems=[pltpu.SemaphoreType.DMA for _ in range(2)],
            idxsem=pltpu.SemaphoreType.DMA,
        ),
        compiler_params=pltpu.CompilerParams(
            needs_layout_passes=False, disable_bounds_checks=True),
    )
    out_main = sc_fill(xf, idxT, addrs)
    return out_main[:, :n_rules]
